# Initial kernel scaffold; baseline (speedup 1.0000x reference)
import jax
import jax.numpy as jnp
from jax.experimental import pallas as pl


def kernel(x, mask):
    raise NotImplementedError("write your pallas kernel here")


# speedup vs baseline: 4.4931x; 4.4931x over previous
import math

import jax
import jax.numpy as jnp
from jax import lax
from jax.experimental import pallas as pl
from jax.experimental.pallas import tpu as pltpu

_NUM_POS_FEATS = 128
_TEMPERATURE = 10000.0
_LOG_T = math.log(_TEMPERATURE)
_ROWS = 512


def _body(mask_ref, o_ref, carry_ref):
    g = pl.program_id(0)

    @pl.when(g == 0)
    def _():
        carry_ref[0] = 0

    carry = carry_ref[0]
    m_row = mask_ref[...].astype(jnp.float32)

    r = _ROWS
    row = lax.broadcasted_iota(jnp.int32, (r, r), 0)
    col = lax.broadcasted_iota(jnp.int32, (r, r), 1)
    tril = (col <= row).astype(jnp.float32)
    local = jnp.sum(tril * m_row, axis=1, keepdims=True)
    vcol = jnp.sum(
        jnp.where(col == row, m_row, 0.0), axis=1, keepdims=True
    )

    rank = local + (carry.astype(jnp.float32) - 1.0)
    carry_ref[0] = carry + jnp.sum(mask_ref[...])

    j = lax.broadcasted_iota(jnp.int32, (1, _NUM_POS_FEATS), 1)
    j2 = (2 * (j // 2)).astype(jnp.float32) * (1.0 / _NUM_POS_FEATS)
    inv_dim = jnp.exp(-j2 * _LOG_T)

    theta = rank * inv_dim
    enc = jnp.where(j % 2 == 0, jnp.sin(theta), jnp.cos(theta))

    o_ref[:, :_NUM_POS_FEATS] = jnp.where(vcol > 0.0, enc, 0.0)
    o_ref[:, _NUM_POS_FEATS:] = jnp.zeros(
        (r, o_ref.shape[1] - _NUM_POS_FEATS), jnp.float32
    )


@jax.jit
def kernel(x, mask):
    bsz, seq_len, feature_dim = x.shape
    grid = seq_len // _ROWS
    out = pl.pallas_call(
        _body,
        grid=(grid,),
        in_specs=[pl.BlockSpec((1, _ROWS), lambda g: (0, g))],
        out_specs=pl.BlockSpec((_ROWS, feature_dim), lambda g: (g, 0)),
        out_shape=jax.ShapeDtypeStruct((seq_len, feature_dim), jnp.float32),
        scratch_shapes=[pltpu.SMEM((1,), jnp.int32)],
        compiler_params=pltpu.CompilerParams(
            dimension_semantics=("arbitrary",),
        ),
    )(mask)
    return out.reshape(bsz, seq_len, feature_dim)
